# baseline (speedup 1.0000x reference)
import functools
import math

import jax
import jax.numpy as jnp
from jax import lax
from jax.experimental import pallas as pl
from jax.experimental.pallas import tpu as pltpu
from jax.experimental.pallas import tpu_sc as plsc

_LANES = 16
_IDX_W = 128


def _embed_sc(tokens_2d, table_slots):
    n_rows, idx_w = tokens_2d.shape
    vocab, slot_w = table_slots.shape
    dim = slot_w // 2
    info = plsc.get_sparse_core_info()
    n_workers = info.num_cores * info.num_subcores
    rows_per_w = n_rows // n_workers
    total = n_rows * idx_w

    mesh = plsc.VectorSubcoreMesh(core_axis_name="c", subcore_axis_name="s")

    @functools.partial(
        pl.kernel,
        mesh=mesh,
        out_type=jax.ShapeDtypeStruct((total, dim), jnp.float32),
        scratch_types=[
            pltpu.VMEM((rows_per_w, idx_w), jnp.int32),
            pltpu.VMEM((idx_w, slot_w), jnp.float32),
            pltpu.VMEM((idx_w, dim), jnp.float32),
            pltpu.SemaphoreType.DMA,
        ],
        compiler_params=pltpu.CompilerParams(use_tc_tiling_on_sc=True),
    )
    def k(tok_hbm, tab_hbm, out_hbm, idx_v, buf_v, obuf_v, sem):
        w = lax.axis_index("s") * info.num_cores + lax.axis_index("c")
        pltpu.sync_copy(tok_hbm.at[pl.ds(w * rows_per_w, rows_per_w)], idx_v)
        tbase = w * rows_per_w * idx_w

        def chunk(j, _):
            pltpu.async_copy(tab_hbm.at[idx_v.at[j]], buf_v, sem).wait()

            def row_body(r, _):
                for k16 in range(dim // _LANES):
                    sl = pl.ds(k16 * _LANES, _LANES)
                    obuf_v[r, sl] = buf_v[r, sl]
                return 0

            lax.fori_loop(0, idx_w, row_body, 0)
            pltpu.sync_copy(obuf_v, out_hbm.at[pl.ds(tbase + j * idx_w, idx_w)])
            return 0

        lax.fori_loop(0, rows_per_w, chunk, 0)

    return k(tokens_2d, table_slots)


def kernel(tokens, embedding_weight):
    b0, b1 = tokens.shape
    vocab, dim = embedding_weight.shape
    scale = math.sqrt(dim)
    toks = tokens.reshape(b0 * b1 // _IDX_W, _IDX_W)
    table_slots = jnp.pad(embedding_weight * scale, ((0, 0), (0, dim)))
    out = _embed_sc(toks, table_slots)
    return out.reshape(b0, b1, dim)
